# Initial kernel scaffold; baseline (speedup 1.0000x reference)
import functools

import jax
import jax.numpy as jnp
from jax import lax
from jax.experimental import pallas as pl
from jax.experimental.pallas import tpu as pltpu


def _topk2_weights(logits, n_exp):
    iota = lax.broadcasted_iota(jnp.int32, logits.shape, 1)
    v1 = jnp.max(logits, axis=1, keepdims=True)
    fi1 = jnp.min(jnp.where(logits == v1, iota, n_exp), axis=1, keepdims=True)
    m1 = iota == fi1
    neg = jnp.where(m1, -jnp.inf, logits)
    v2 = jnp.max(neg, axis=1, keepdims=True)
    fi2 = jnp.min(jnp.where(neg == v2, iota, n_exp), axis=1, keepdims=True)
    m2 = iota == fi2
    e2 = jnp.exp(v2 - v1)
    s1 = 1.0 / (1.0 + e2)
    s2 = e2 * s1
    zero = jnp.zeros_like(logits)
    return jnp.where(m1, s1, zero) + jnp.where(m2, s2, zero)


def _silu(x):
    return x * (1.0 / (1.0 + jnp.exp(-x)))


def _moe_body(x_ref, rw_ref, g_ref, u_ref, d_ref, out_ref, wts_ref, *, n_exp):
    e = pl.program_id(1)
    h = pl.program_id(2)
    first = jnp.logical_and(e == 0, h == 0)

    @pl.when(first)
    def _router():
        logits = jax.lax.dot_general(
            x_ref[...], rw_ref[...], (((1,), (1,)), ((), ())),
            preferred_element_type=jnp.float32)
        wts_ref[...] = _topk2_weights(logits, n_exp)

    x = x_ref[...]
    g = jnp.dot(x, g_ref[0], preferred_element_type=jnp.float32)
    u = jnp.dot(x, u_ref[0], preferred_element_type=jnp.float32)
    o = jnp.dot(_silu(g) * u, d_ref[0], preferred_element_type=jnp.float32)
    contrib = wts_ref[:, e][:, None] * o

    @pl.when(first)
    def _init():
        out_ref[...] = contrib

    @pl.when(jnp.logical_not(first))
    def _acc():
        out_ref[...] += contrib


def _ffn_body(x_ref, w1_ref, w3_ref, w2_ref, moe_ref, out_ref):
    s = pl.program_id(1)
    x = x_ref[...]
    a = jax.lax.dot_general(x, w1_ref[...], (((1,), (1,)), ((), ())),
                            preferred_element_type=jnp.float32)
    b = jax.lax.dot_general(x, w3_ref[...], (((1,), (1,)), ((), ())),
                            preferred_element_type=jnp.float32)
    hblk = _silu(a) * b
    o = jax.lax.dot_general(hblk, w2_ref[...], (((1,), (1,)), ((), ())),
                            preferred_element_type=jnp.float32)

    @pl.when(s == 0)
    def _init():
        out_ref[...] = moe_ref[...] + o

    @pl.when(s != 0)
    def _acc():
        out_ref[...] += o


def kernel(x, router_w, gate_proj, up_proj, down_proj, w1, w2, w3):
    T, D = x.shape
    E, _, H = gate_proj.shape
    SH = w1.shape[0]

    TM = min(T, 2048)
    HC = min(H, 512)
    SC = min(SH, 512)
    t2, h2, s2 = T // TM, H // HC, SH // SC

    moe_out = pl.pallas_call(
        functools.partial(_moe_body, n_exp=E),
        grid=(t2, E, h2),
        in_specs=[
            pl.BlockSpec((TM, D), lambda t, e, h: (t, 0)),
            pl.BlockSpec((E, D), lambda t, e, h: (0, 0)),
            pl.BlockSpec((1, D, HC), lambda t, e, h: (e, 0, h)),
            pl.BlockSpec((1, D, HC), lambda t, e, h: (e, 0, h)),
            pl.BlockSpec((1, HC, D), lambda t, e, h: (e, h, 0)),
        ],
        out_specs=pl.BlockSpec((TM, D), lambda t, e, h: (t, 0)),
        out_shape=jax.ShapeDtypeStruct((T, D), jnp.float32),
        scratch_shapes=[pltpu.VMEM((TM, E), jnp.float32)],
    )(x, router_w, gate_proj, up_proj, down_proj)

    out = pl.pallas_call(
        _ffn_body,
        grid=(t2, s2),
        in_specs=[
            pl.BlockSpec((TM, D), lambda t, s: (t, 0)),
            pl.BlockSpec((SC, D), lambda t, s: (s, 0)),
            pl.BlockSpec((SC, D), lambda t, s: (s, 0)),
            pl.BlockSpec((D, SC), lambda t, s: (0, s)),
            pl.BlockSpec((TM, D), lambda t, s: (t, 0)),
        ],
        out_specs=pl.BlockSpec((TM, D), lambda t, s: (t, 0)),
        out_shape=jax.ShapeDtypeStruct((T, D), jnp.float32),
    )(x, w1, w3, w2, moe_out)

    return out


# speedup vs baseline: 1.1986x; 1.1986x over previous
import functools

import jax
import jax.numpy as jnp
from jax import lax
from jax.experimental import pallas as pl
from jax.experimental.pallas import tpu as pltpu


def _topk2_weights(logits, n_exp):
    iota = lax.broadcasted_iota(jnp.int32, logits.shape, 1)
    v1 = jnp.max(logits, axis=1, keepdims=True)
    fi1 = jnp.min(jnp.where(logits == v1, iota, n_exp), axis=1, keepdims=True)
    m1 = iota == fi1
    neg = jnp.where(m1, -jnp.inf, logits)
    v2 = jnp.max(neg, axis=1, keepdims=True)
    fi2 = jnp.min(jnp.where(neg == v2, iota, n_exp), axis=1, keepdims=True)
    m2 = iota == fi2
    e2 = jnp.exp(v2 - v1)
    s1 = 1.0 / (1.0 + e2)
    s2 = e2 * s1
    zero = jnp.zeros_like(logits)
    return jnp.where(m1, s1, zero) + jnp.where(m2, s2, zero)


def _silu(x):
    return x * (1.0 / (1.0 + jnp.exp(-x)))


def _moe_body(x_ref, rw_ref, g_ref, u_ref, d_ref, out_ref, wts_ref, *, n_exp):
    e = pl.program_id(1)
    h = pl.program_id(2)
    first = jnp.logical_and(e == 0, h == 0)

    @pl.when(first)
    def _router():
        logits = jax.lax.dot_general(
            x_ref[...], rw_ref[...], (((1,), (1,)), ((), ())),
            preferred_element_type=jnp.float32)
        wts_ref[...] = _topk2_weights(logits, n_exp)

    x = x_ref[...]
    g = jnp.dot(x, g_ref[0], preferred_element_type=jnp.float32)
    u = jnp.dot(x, u_ref[0], preferred_element_type=jnp.float32)
    o = jnp.dot(_silu(g) * u, d_ref[0], preferred_element_type=jnp.float32)
    wts = wts_ref[...]
    col = lax.broadcasted_iota(jnp.int32, wts.shape, 1) == e
    wcol = jnp.sum(jnp.where(col, wts, 0.0), axis=1, keepdims=True)
    contrib = wcol * o

    @pl.when(first)
    def _init():
        out_ref[...] = contrib

    @pl.when(jnp.logical_not(first))
    def _acc():
        out_ref[...] += contrib


def _ffn_body(x_ref, w1_ref, w3_ref, w2_ref, moe_ref, out_ref):
    s = pl.program_id(1)
    x = x_ref[...]
    a = jax.lax.dot_general(x, w1_ref[...], (((1,), (1,)), ((), ())),
                            preferred_element_type=jnp.float32)
    b = jax.lax.dot_general(x, w3_ref[...], (((1,), (1,)), ((), ())),
                            preferred_element_type=jnp.float32)
    hblk = _silu(a) * b
    o = jax.lax.dot_general(hblk, w2_ref[...], (((1,), (1,)), ((), ())),
                            preferred_element_type=jnp.float32)

    @pl.when(s == 0)
    def _init():
        out_ref[...] = moe_ref[...] + o

    @pl.when(s != 0)
    def _acc():
        out_ref[...] += o


def kernel(x, router_w, gate_proj, up_proj, down_proj, w1, w2, w3):
    T, D = x.shape
    E, _, H = gate_proj.shape
    SH = w1.shape[0]

    TM = min(T, 1024)
    HC = min(H, 256)
    TMF = min(T, 512)
    SC = min(SH, 512)
    t2, h2 = T // TM, H // HC
    tf2, s2 = T // TMF, SH // SC

    moe_out = pl.pallas_call(
        functools.partial(_moe_body, n_exp=E),
        grid=(t2, E, h2),
        in_specs=[
            pl.BlockSpec((TM, D), lambda t, e, h: (t, 0)),
            pl.BlockSpec((E, D), lambda t, e, h: (0, 0)),
            pl.BlockSpec((1, D, HC), lambda t, e, h: (e, 0, h)),
            pl.BlockSpec((1, D, HC), lambda t, e, h: (e, 0, h)),
            pl.BlockSpec((1, HC, D), lambda t, e, h: (e, h, 0)),
        ],
        out_specs=pl.BlockSpec((TM, D), lambda t, e, h: (t, 0)),
        out_shape=jax.ShapeDtypeStruct((T, D), jnp.float32),
        scratch_shapes=[pltpu.VMEM((TM, E), jnp.float32)],
    )(x, router_w, gate_proj, up_proj, down_proj)

    out = pl.pallas_call(
        _ffn_body,
        grid=(tf2, s2),
        in_specs=[
            pl.BlockSpec((TMF, D), lambda t, s: (t, 0)),
            pl.BlockSpec((SC, D), lambda t, s: (s, 0)),
            pl.BlockSpec((SC, D), lambda t, s: (s, 0)),
            pl.BlockSpec((D, SC), lambda t, s: (0, s)),
            pl.BlockSpec((TMF, D), lambda t, s: (t, 0)),
        ],
        out_specs=pl.BlockSpec((TMF, D), lambda t, s: (t, 0)),
        out_shape=jax.ShapeDtypeStruct((T, D), jnp.float32),
    )(x, w1, w3, w2, moe_out)

    return out
